# Initial kernel scaffold; baseline (speedup 1.0000x reference)
import jax
import jax.numpy as jnp
from jax.experimental import pallas as pl


def kernel(input_ids, token_table, pos_table):
    raise NotImplementedError("write your pallas kernel here")


# speedup vs baseline: 1.2182x; 1.2182x over previous
import jax
import jax.numpy as jnp
from jax import lax
from jax.experimental import pallas as pl
from jax.experimental.pallas import tpu as pltpu
from jax.experimental.pallas import tpu_sc as plsc

D = 1024
SEQ = 4096
LANES = 16
NC = 2
NS = 16
NW = NC * NS
ROWS_PER_W = 512
CHUNK = 32
N_CHUNKS = ROWS_PER_W // CHUNK
VREGS_PER_ROW = D // LANES


def _emb_body(ids_hbm, tok_hbm, pos_hbm, out_hbm, idx_v, acc_v, tok_v, sem):
    wid = lax.axis_index("s") * NC + lax.axis_index("c")
    base = wid * ROWS_PER_W
    pos_base = base % SEQ
    pltpu.sync_copy(ids_hbm.at[pl.ds(base, ROWS_PER_W)], idx_v)

    def chunk_body(j, carry):
        row0 = j * CHUNK
        pltpu.sync_copy(pos_hbm.at[pl.ds(pos_base + row0, CHUNK)], acc_v)
        pltpu.async_copy(tok_hbm.at[idx_v.at[pl.ds(row0, CHUNK)]], tok_v,
                         sem).wait()

        def add_row(r, c2):
            for l in range(VREGS_PER_ROW):
                sl = pl.ds(l * LANES, LANES)
                plsc.addupdate(acc_v.at[r, sl], tok_v[r, sl])
            return c2

        lax.fori_loop(0, CHUNK, add_row, 0)
        pltpu.sync_copy(acc_v, out_hbm.at[pl.ds(base + row0, CHUNK)])
        return carry

    lax.fori_loop(0, N_CHUNKS, chunk_body, 0)


def kernel(input_ids, token_table, pos_table):
    b, s = input_ids.shape
    ids_flat = input_ids.reshape(-1).astype(jnp.int32)
    k = pl.kernel(
        _emb_body,
        mesh=plsc.VectorSubcoreMesh(core_axis_name="c", subcore_axis_name="s"),
        out_type=jax.ShapeDtypeStruct((b * s, D), jnp.float32),
        scratch_types=[
            pltpu.VMEM((ROWS_PER_W,), jnp.int32),
            pltpu.VMEM((CHUNK, D), jnp.float32),
            pltpu.VMEM((CHUNK, D), jnp.float32),
            pltpu.SemaphoreType.DMA,
        ],
    )
    out = k(ids_flat, token_table, pos_table)
    return out.reshape(b, s, D)


# speedup vs baseline: 1.2776x; 1.0488x over previous
import jax
import jax.numpy as jnp
from jax import lax
from jax.experimental import pallas as pl
from jax.experimental.pallas import tpu as pltpu
from jax.experimental.pallas import tpu_sc as plsc

D = 1024
SEQ = 4096
LANES = 16
NC = 2
NS = 16
NW = NC * NS
ROWS_PER_W = 512
CHUNK = 32
N_CHUNKS = ROWS_PER_W // CHUNK
VREGS_PER_ROW = D // LANES


def _emb_body(ids_hbm, tok_hbm, pos_hbm, out_hbm,
              idx_v, tok0_v, tok1_v, pos_v, sem_g, sem_w0, sem_w1):
    wid = lax.axis_index("s") * NC + lax.axis_index("c")
    base = wid * ROWS_PER_W
    pos_base = base % SEQ
    pltpu.sync_copy(ids_hbm.at[pl.ds(base, ROWS_PER_W)], idx_v)

    toks = (tok0_v, tok1_v)
    wsems = (sem_w0, sem_w1)

    def gather_desc(j, buf):
        return pltpu.make_async_copy(
            tok_hbm.at[idx_v.at[pl.ds(j * CHUNK, CHUNK)]], buf, sem_g)

    def wb_desc(j, buf):
        return pltpu.make_async_copy(
            buf, out_hbm.at[pl.ds(base + j * CHUNK, CHUNK)], wsems[j % 2])

    gather_desc(0, toks[0]).start()
    for j in range(N_CHUNKS):
        b = j % 2
        gather_desc(j, toks[b]).wait()
        if j + 1 < N_CHUNKS:
            if j >= 1:
                wb_desc(j - 1, toks[(j + 1) % 2]).wait()
            gather_desc(j + 1, toks[(j + 1) % 2]).start()
        pltpu.sync_copy(pos_hbm.at[pl.ds(pos_base + j * CHUNK, CHUNK)], pos_v)

        def add_row(r, c2, _tok=toks[b]):
            for l in range(VREGS_PER_ROW):
                sl = pl.ds(l * LANES, LANES)
                plsc.addupdate(_tok.at[r, sl], pos_v[r, sl])
            return c2

        lax.fori_loop(0, CHUNK, add_row, 0)
        wb_desc(j, toks[b]).start()

    wb_desc(N_CHUNKS - 2, toks[(N_CHUNKS - 2) % 2]).wait()
    wb_desc(N_CHUNKS - 1, toks[(N_CHUNKS - 1) % 2]).wait()


def kernel(input_ids, token_table, pos_table):
    b, s = input_ids.shape
    ids_flat = input_ids.reshape(-1).astype(jnp.int32)
    k = pl.kernel(
        _emb_body,
        mesh=plsc.VectorSubcoreMesh(core_axis_name="c", subcore_axis_name="s"),
        out_type=jax.ShapeDtypeStruct((b * s, D), jnp.float32),
        scratch_types=[
            pltpu.VMEM((ROWS_PER_W,), jnp.int32),
            pltpu.VMEM((CHUNK, D), jnp.float32),
            pltpu.VMEM((CHUNK, D), jnp.float32),
            pltpu.VMEM((CHUNK, D), jnp.float32),
            pltpu.SemaphoreType.DMA,
            pltpu.SemaphoreType.DMA,
            pltpu.SemaphoreType.DMA,
        ],
    )
    out = k(ids_flat, token_table, pos_table)
    return out.reshape(b, s, D)


# speedup vs baseline: 1.7186x; 1.3451x over previous
import jax
import jax.numpy as jnp
from jax import lax
from jax.experimental import pallas as pl
from jax.experimental.pallas import tpu as pltpu
from jax.experimental.pallas import tpu_sc as plsc

D = 1024
SEQ = 4096
BATCH = 4
LANES = 16
NC = 2
NS = 16
NW = NC * NS
POS_PER_W = SEQ // NW
CHUNK = 32
N_CHUNKS = POS_PER_W // CHUNK
N_STEPS = N_CHUNKS * BATCH
VREGS_PER_ROW = D // LANES


def _emb_body(ids_hbm, tok_hbm, pos_hbm, out_hbm,
              idx_v, tok0_v, tok1_v, pos_v, sem_g, sem_w0, sem_w1):
    wid = lax.axis_index("s") * NC + lax.axis_index("c")
    pos0 = wid * POS_PER_W
    for b in range(BATCH):
        pltpu.sync_copy(ids_hbm.at[pl.ds(b * SEQ + pos0, POS_PER_W)],
                        idx_v.at[pl.ds(b * POS_PER_W, POS_PER_W)])

    toks = (tok0_v, tok1_v)
    wsems = (sem_w0, sem_w1)

    def gather_desc(t, buf):
        c, b = t // BATCH, t % BATCH
        sl = pl.ds(b * POS_PER_W + c * CHUNK, CHUNK)
        return pltpu.make_async_copy(tok_hbm.at[idx_v.at[sl]], buf, sem_g)

    def wb_desc(t, buf):
        c, b = t // BATCH, t % BATCH
        sl = pl.ds(b * SEQ + pos0 + c * CHUNK, CHUNK)
        return pltpu.make_async_copy(buf, out_hbm.at[sl], wsems[t % 2])

    gather_desc(0, toks[0]).start()
    for t in range(N_STEPS):
        c, b = t // BATCH, t % BATCH
        gather_desc(t, toks[t % 2]).wait()
        if t + 1 < N_STEPS:
            if t >= 1:
                wb_desc(t - 1, toks[(t + 1) % 2]).wait()
            gather_desc(t + 1, toks[(t + 1) % 2]).start()
        if b == 0:
            pltpu.sync_copy(pos_hbm.at[pl.ds(pos0 + c * CHUNK, CHUNK)], pos_v)

        def add_row(r, c2, _tok=toks[t % 2]):
            for l in range(VREGS_PER_ROW):
                sl = pl.ds(l * LANES, LANES)
                plsc.addupdate(_tok.at[r, sl], pos_v[r, sl])
            return c2

        lax.fori_loop(0, CHUNK, add_row, 0)
        wb_desc(t, toks[t % 2]).start()

    wb_desc(N_STEPS - 2, toks[(N_STEPS - 2) % 2]).wait()
    wb_desc(N_STEPS - 1, toks[(N_STEPS - 1) % 2]).wait()


def kernel(input_ids, token_table, pos_table):
    b, s = input_ids.shape
    ids_flat = input_ids.reshape(-1).astype(jnp.int32)
    k = pl.kernel(
        _emb_body,
        mesh=plsc.VectorSubcoreMesh(core_axis_name="c", subcore_axis_name="s"),
        out_type=jax.ShapeDtypeStruct((b * s, D), jnp.float32),
        scratch_types=[
            pltpu.VMEM((BATCH * POS_PER_W,), jnp.int32),
            pltpu.VMEM((CHUNK, D), jnp.float32),
            pltpu.VMEM((CHUNK, D), jnp.float32),
            pltpu.VMEM((CHUNK, D), jnp.float32),
            pltpu.SemaphoreType.DMA,
            pltpu.SemaphoreType.DMA,
            pltpu.SemaphoreType.DMA,
        ],
    )
    out = k(ids_flat, token_table, pos_table)
    return out.reshape(b, s, D)
